# baseline (speedup 1.0000x reference)
import dataclasses
import functools

import jax
import jax.numpy as jnp
from jax import lax
from jax.experimental import pallas as pl
from jax.experimental.pallas import tpu as pltpu
from jax.experimental.pallas import tpu_sc as plsc

N_CORES = 2
N_SUBCORES = 16
NW = N_CORES * N_SUBCORES
LANES = 16
CHUNK = 512
EPC = 256
SUP = 2048
CPS = SUP // EPC
NBUF = 8
TPAD = 640
REC = 8


def _rsqrt_f32(s):
    i = plsc.bitcast(s, jnp.int32)
    i = jnp.int32(0x5F3759DF) - lax.shift_right_logical(i, 1)
    y = plsc.bitcast(i, jnp.float32)
    half_s = s * 0.5
    for _ in range(3):
        y = y * (1.5 - half_s * y * y)
    return y


@functools.lru_cache(maxsize=None)
def _build_sc_kernel(n_edges: int):
    nsup = n_edges // SUP
    nper = (nsup + NW - 1) // NW
    npair = (nper + 1) // 2

    mesh = plsc.VectorSubcoreMesh(core_axis_name="c", subcore_axis_name="s")
    cp = pltpu.CompilerParams()
    if "needs_layout_passes" in pltpu.CompilerParams.__dataclass_fields__:
        cp = dataclasses.replace(cp, needs_layout_passes=False)
    if "use_tc_tiling_on_sc" in pltpu.CompilerParams.__dataclass_fields__:
        cp = dataclasses.replace(cp, use_tc_tiling_on_sc=False)

    @functools.partial(
        pl.kernel,
        out_type=jax.ShapeDtypeStruct((n_edges,), jnp.float32),
        mesh=mesh,
        compiler_params=cp,
        scratch_types=[
            pltpu.VMEM((2 * 2 * SUP,), jnp.int32),
            pltpu.VMEM((NBUF * CHUNK, REC), jnp.float32),
            pltpu.VMEM((2 * SUP,), jnp.float32),
            pltpu.VMEM((TPAD,), jnp.float32),
            pltpu.VMEM((TPAD,), jnp.float32),
            pltpu.SemaphoreType.DMA((2,)),
            pltpu.SemaphoreType.DMA((NBUF,)),
            pltpu.SemaphoreType.DMA((2,)),
        ],
    )
    def sc_kernel(nodes_hbm, map_hbm, x0_hbm, k_hbm, out_hbm,
                  idx_v, rec_v, out_v, x0_v, k_v,
                  sem_i, sem_r, sem_o):
        wid = lax.axis_index("s") * N_CORES + lax.axis_index("c")
        pltpu.sync_copy(x0_hbm, x0_v)
        pltpu.sync_copy(k_hbm, k_v)

        lane_iota = lax.iota(jnp.int32, LANES)
        cols = [jnp.full((LANES,), c, jnp.int32) for c in range(4)]

        NIX = 2 * SUP

        def fire_idx(blk, slot):
            pltpu.async_copy(map_hbm.at[pl.ds(blk * NIX, NIX)],
                             idx_v.at[pl.ds(slot * NIX, NIX)],
                             sem_i.at[slot])

        def wait_idx(slot):
            pltpu.make_async_copy(map_hbm.at[pl.ds(0, NIX)],
                                  idx_v.at[pl.ds(slot * NIX, NIX)],
                                  sem_i.at[slot]).wait()

        HC = CHUNK // 2

        def fire_gather(slot, row, buf):
            pltpu.async_copy(
                nodes_hbm.at[idx_v.at[pl.ds(slot * NIX + row * CHUNK, HC)]],
                rec_v.at[pl.ds(buf * CHUNK, HC)], sem_r.at[buf])
            pltpu.async_copy(
                nodes_hbm.at[idx_v.at[pl.ds(slot * NIX + row * CHUNK + HC,
                                            HC)]],
                rec_v.at[pl.ds(buf * CHUNK + HC, HC)], sem_r.at[buf])

        def wait_gather(buf):
            pltpu.make_async_copy(nodes_hbm.at[idx_v.at[pl.ds(0, HC)]],
                                  rec_v.at[pl.ds(buf * CHUNK, HC)],
                                  sem_r.at[buf]).wait()
            pltpu.make_async_copy(nodes_hbm.at[idx_v.at[pl.ds(0, HC)]],
                                  rec_v.at[pl.ds(buf * CHUNK + HC, HC)],
                                  sem_r.at[buf]).wait()

        def compute_chunk(slot, c, buf):
            for h in range(2):
                base = buf * CHUNK + h * 256

                @pl.loop(0, 128, step=LANES)
                def _(j):
                    rows = base + j + lane_iota
                    rows_d = rows + 128
                    sx = plsc.load_gather(rec_v, [rows, cols[0]])
                    sy = plsc.load_gather(rec_v, [rows, cols[1]])
                    sz = plsc.load_gather(rec_v, [rows, cols[2]])
                    st = plsc.load_gather(rec_v, [rows, cols[3]])
                    dx_ = plsc.load_gather(rec_v, [rows_d, cols[0]])
                    dy_ = plsc.load_gather(rec_v, [rows_d, cols[1]])
                    dz_ = plsc.load_gather(rec_v, [rows_d, cols[2]])
                    dt = plsc.load_gather(rec_v, [rows_d, cols[3]])
                    ex = dx_ - sx
                    ey = dy_ - sy
                    ez = dz_ - sz
                    s = ex * ex + ey * ey + ez * ez + 1e-12
                    d = s * _rsqrt_f32(s)
                    t0 = st.astype(jnp.int32)
                    t1 = dt.astype(jnp.int32)
                    pidx = t0 * 25 + t1
                    x0 = plsc.load_gather(x0_v, [pidx])
                    kk = plsc.load_gather(k_v, [pidx])
                    u = d - x0
                    out_v[pl.ds(slot * SUP + c * EPC + h * 128 + j,
                                LANES)] = kk * u * u

        def do_super(blk, blk_is_valid, slot):
            nxt = blk + NW

            @pl.when(blk_is_valid)
            def _():
                @pl.when((blk - 2 * NW >= 0) & (blk - 2 * NW < nsup))
                def _():
                    pltpu.make_async_copy(
                        out_v.at[pl.ds(slot * SUP, SUP)],
                        out_hbm.at[pl.ds(0, SUP)], sem_o.at[slot]).wait()

                nxt_valid = nxt < nsup
                for c in range(CPS):
                    buf = c % NBUF
                    wait_gather(buf)
                    compute_chunk(slot, c, buf)
                    if c + NBUF < CPS:
                        fire_gather(slot, c + NBUF, buf)
                    else:
                        if c + NBUF == CPS:
                            @pl.when(nxt_valid)
                            def _():
                                wait_idx(1 - slot)

                        @pl.when(nxt_valid)
                        def _():
                            fire_gather(1 - slot, c + NBUF - CPS, buf)

                pltpu.async_copy(out_v.at[pl.ds(slot * SUP, SUP)],
                                 out_hbm.at[pl.ds(blk * SUP, SUP)],
                                 sem_o.at[slot])
                @pl.when(blk + 2 * NW < nsup)
                def _():
                    fire_idx(blk + 2 * NW, slot)

        fire_idx(wid, 0)
        fire_idx(wid + NW, 1)
        wait_idx(0)
        for b in range(NBUF):
            fire_gather(0, b, b)

        @pl.loop(0, npair)
        def _(p):
            i0 = 2 * p
            blk0 = i0 * NW + wid
            do_super(blk0, blk0 < nsup, 0)
            blk1 = (i0 + 1) * NW + wid
            do_super(blk1, blk1 < nsup, 1)

        for slot in range(2):
            pltpu.make_async_copy(out_v.at[pl.ds(slot * SUP, SUP)],
                                  out_hbm.at[pl.ds(0, SUP)],
                                  sem_o.at[slot]).wait()

    return sc_kernel


def kernel(pos, mapping, atom_types, x0_table, k_table):
    n_edges = mapping.shape[1]
    t_f = atom_types.astype(jnp.float32)
    nodes = jnp.concatenate(
        [pos, t_f[:, None],
         jnp.zeros((pos.shape[0], REC - 4), jnp.float32)], axis=1)
    x0f = jnp.zeros((TPAD,), jnp.float32).at[:625].set(x0_table.reshape(-1))
    kf = jnp.zeros((TPAD,), jnp.float32).at[:625].set(k_table.reshape(-1))
    map_il = (mapping.reshape(2, n_edges // 128, 128)
              .transpose(1, 0, 2).reshape(-1))
    return _build_sc_kernel(n_edges)(nodes, map_il, x0f, kf)


# speedup vs baseline: 1.2522x; 1.2522x over previous
import dataclasses
import functools

import jax
import jax.numpy as jnp
from jax import lax
from jax.experimental import pallas as pl
from jax.experimental.pallas import tpu as pltpu
from jax.experimental.pallas import tpu_sc as plsc

N_CORES = 2
N_SUBCORES = 16
NW = N_CORES * N_SUBCORES
LANES = 16
CHUNK = 512
SUP = 2048
CPS = SUP // CHUNK
NBUF = 4
TPAD = 640
REC = 8


def _rsqrt_f32(s):
    i = plsc.bitcast(s, jnp.int32)
    i = jnp.int32(0x5F3759DF) - lax.shift_right_logical(i, 1)
    y = plsc.bitcast(i, jnp.float32)
    half_s = s * 0.5
    for _ in range(3):
        y = y * (1.5 - half_s * y * y)
    return y


@functools.lru_cache(maxsize=None)
def _build_sc_kernel(n_edges: int):
    nsup = n_edges // SUP
    nper = (nsup + NW - 1) // NW
    npair = (nper + 1) // 2

    mesh = plsc.VectorSubcoreMesh(core_axis_name="c", subcore_axis_name="s")
    cp = pltpu.CompilerParams()
    if "needs_layout_passes" in pltpu.CompilerParams.__dataclass_fields__:
        cp = dataclasses.replace(cp, needs_layout_passes=False)
    if "use_tc_tiling_on_sc" in pltpu.CompilerParams.__dataclass_fields__:
        cp = dataclasses.replace(cp, use_tc_tiling_on_sc=False)

    @functools.partial(
        pl.kernel,
        out_type=jax.ShapeDtypeStruct((n_edges,), jnp.float32),
        mesh=mesh,
        compiler_params=cp,
        scratch_types=[
            pltpu.VMEM((2 * SUP,), jnp.int32),
            pltpu.VMEM((2 * SUP,), jnp.int32),
            pltpu.VMEM((NBUF * CHUNK, REC), jnp.float32),
            pltpu.VMEM((NBUF * CHUNK, REC), jnp.float32),
            pltpu.VMEM((2 * SUP,), jnp.float32),
            pltpu.VMEM((TPAD,), jnp.float32),
            pltpu.VMEM((TPAD,), jnp.float32),
            pltpu.SemaphoreType.DMA((2,)),
            pltpu.SemaphoreType.DMA((NBUF,)),
            pltpu.SemaphoreType.DMA((2,)),
        ],
    )
    def sc_kernel(nodes_hbm, map_hbm, x0_hbm, k_hbm, out_hbm,
                  sidx_v, didx_v, srec_v, drec_v, out_v, x0_v, k_v,
                  sem_i, sem_r, sem_o):
        wid = lax.axis_index("s") * N_CORES + lax.axis_index("c")
        pltpu.sync_copy(x0_hbm, x0_v)
        pltpu.sync_copy(k_hbm, k_v)

        lane_iota = lax.iota(jnp.int32, LANES)
        cols = [jnp.full((LANES,), c, jnp.int32) for c in range(4)]

        def fire_idx(blk, slot):
            pltpu.async_copy(map_hbm.at[0, pl.ds(blk * SUP, SUP)],
                             sidx_v.at[pl.ds(slot * SUP, SUP)], sem_i.at[slot])
            pltpu.async_copy(map_hbm.at[1, pl.ds(blk * SUP, SUP)],
                             didx_v.at[pl.ds(slot * SUP, SUP)], sem_i.at[slot])

        def wait_idx(slot):
            pltpu.make_async_copy(map_hbm.at[0, pl.ds(0, SUP)],
                                  sidx_v.at[pl.ds(slot * SUP, SUP)],
                                  sem_i.at[slot]).wait()
            pltpu.make_async_copy(map_hbm.at[1, pl.ds(0, SUP)],
                                  didx_v.at[pl.ds(slot * SUP, SUP)],
                                  sem_i.at[slot]).wait()

        def fire_gather(slot, row, buf):
            pltpu.async_copy(
                nodes_hbm.at[sidx_v.at[pl.ds(slot * SUP + row * CHUNK, CHUNK)]],
                srec_v.at[pl.ds(buf * CHUNK, CHUNK)], sem_r.at[buf])
            pltpu.async_copy(
                nodes_hbm.at[didx_v.at[pl.ds(slot * SUP + row * CHUNK, CHUNK)]],
                drec_v.at[pl.ds(buf * CHUNK, CHUNK)], sem_r.at[buf])

        def wait_gather(buf):
            pltpu.make_async_copy(nodes_hbm.at[sidx_v.at[pl.ds(0, CHUNK)]],
                                  srec_v.at[pl.ds(buf * CHUNK, CHUNK)],
                                  sem_r.at[buf]).wait()
            pltpu.make_async_copy(nodes_hbm.at[sidx_v.at[pl.ds(0, CHUNK)]],
                                  drec_v.at[pl.ds(buf * CHUNK, CHUNK)],
                                  sem_r.at[buf]).wait()

        def compute_chunk(slot, c, buf):
            base = buf * CHUNK

            @pl.loop(0, CHUNK, step=LANES)
            def _(j):
                rows = base + j + lane_iota
                sx = plsc.load_gather(srec_v, [rows, cols[0]])
                sy = plsc.load_gather(srec_v, [rows, cols[1]])
                sz = plsc.load_gather(srec_v, [rows, cols[2]])
                st = plsc.load_gather(srec_v, [rows, cols[3]])
                dx_ = plsc.load_gather(drec_v, [rows, cols[0]])
                dy_ = plsc.load_gather(drec_v, [rows, cols[1]])
                dz_ = plsc.load_gather(drec_v, [rows, cols[2]])
                dt = plsc.load_gather(drec_v, [rows, cols[3]])
                ex = dx_ - sx
                ey = dy_ - sy
                ez = dz_ - sz
                s = ex * ex + ey * ey + ez * ez + 1e-12
                d = s * _rsqrt_f32(s)
                t0 = st.astype(jnp.int32)
                t1 = dt.astype(jnp.int32)
                pidx = t0 * 25 + t1
                x0 = plsc.load_gather(x0_v, [pidx])
                kk = plsc.load_gather(k_v, [pidx])
                u = d - x0
                out_v[pl.ds(slot * SUP + c * CHUNK + j, LANES)] = kk * u * u

        def do_super(blk, blk_is_valid, slot):
            nxt = blk + NW

            @pl.when(blk_is_valid)
            def _():
                @pl.when((blk - 2 * NW >= 0) & (blk - 2 * NW < nsup))
                def _():
                    pltpu.make_async_copy(
                        out_v.at[pl.ds(slot * SUP, SUP)],
                        out_hbm.at[pl.ds(0, SUP)], sem_o.at[slot]).wait()

                nxt_valid = nxt < nsup
                for c in range(CPS):
                    buf = c % NBUF
                    wait_gather(buf)
                    compute_chunk(slot, c, buf)
                    if c + NBUF < CPS:
                        fire_gather(slot, c + NBUF, buf)
                    else:
                        if c + NBUF == CPS:
                            @pl.when(nxt_valid)
                            def _():
                                wait_idx(1 - slot)

                        @pl.when(nxt_valid)
                        def _():
                            fire_gather(1 - slot, c + NBUF - CPS, buf)

                pltpu.async_copy(out_v.at[pl.ds(slot * SUP, SUP)],
                                 out_hbm.at[pl.ds(blk * SUP, SUP)],
                                 sem_o.at[slot])
                @pl.when(blk + 2 * NW < nsup)
                def _():
                    fire_idx(blk + 2 * NW, slot)

        fire_idx(wid, 0)
        fire_idx(wid + NW, 1)
        wait_idx(0)
        for b in range(NBUF):
            fire_gather(0, b, b)

        @pl.loop(0, npair)
        def _(p):
            i0 = 2 * p
            blk0 = i0 * NW + wid
            do_super(blk0, blk0 < nsup, 0)
            blk1 = (i0 + 1) * NW + wid
            do_super(blk1, blk1 < nsup, 1)

        for slot in range(2):
            pltpu.make_async_copy(out_v.at[pl.ds(slot * SUP, SUP)],
                                  out_hbm.at[pl.ds(0, SUP)],
                                  sem_o.at[slot]).wait()

    return sc_kernel


def kernel(pos, mapping, atom_types, x0_table, k_table):
    n_edges = mapping.shape[1]
    t_f = atom_types.astype(jnp.float32)
    nodes = jnp.concatenate(
        [pos, t_f[:, None],
         jnp.zeros((pos.shape[0], REC - 4), jnp.float32)], axis=1)
    x0f = jnp.zeros((TPAD,), jnp.float32).at[:625].set(x0_table.reshape(-1))
    kf = jnp.zeros((TPAD,), jnp.float32).at[:625].set(k_table.reshape(-1))
    return _build_sc_kernel(n_edges)(nodes, mapping, x0f, kf)
